# baseline (speedup 1.0000x reference)
import jax
import jax.numpy as jnp
from jax.experimental import pallas as pl
from jax.experimental.pallas import tpu as pltpu

NCLS = 24
LANES = 128


def _bin(kl, lab, sums_ref, cnts_ref):
    s = kl.shape[1]
    cls = jax.lax.broadcasted_iota(jnp.int32, (NCLS, 1), 0)
    mask = lab == cls
    contrib = jnp.where(mask, kl, jnp.float32(0.0))
    cnt = mask.astype(jnp.float32)
    part_s = jnp.zeros((NCLS, LANES), jnp.float32)
    part_c = jnp.zeros((NCLS, LANES), jnp.float32)
    for j in range(s // LANES):
        part_s = part_s + contrib[:, j * LANES:(j + 1) * LANES]
        part_c = part_c + cnt[:, j * LANES:(j + 1) * LANES]
    sums_ref[...] += part_s
    cnts_ref[...] += part_c


def _body(gate_ref, x0_ref, y0_ref, lab0_ref, x1_ref, y1_ref, lab1_ref,
          out_ref, s0_ref, c0_ref, s1_ref, c1_ref):
    i = pl.program_id(0)

    @pl.when(i == 0)
    def _init():
        s0_ref[...] = jnp.zeros_like(s0_ref)
        c0_ref[...] = jnp.zeros_like(c0_ref)
        s1_ref[...] = jnp.zeros_like(s1_ref)
        c1_ref[...] = jnp.zeros_like(c1_ref)

    @pl.when(i < 8)
    def _scale0():
        x = x0_ref[0]
        y = y0_ref[0]
        mx = jnp.max(x, axis=0, keepdims=True)
        ex = jnp.exp(x - mx)
        sx = jnp.sum(ex, axis=0, keepdims=True)
        my = jnp.max(y, axis=0, keepdims=True)
        ey = jnp.exp(y - my)
        sy = jnp.sum(ey, axis=0, keepdims=True)
        t = jnp.sum(ex * (x - y), axis=0, keepdims=True) / sx
        kl = t - (mx + jnp.log(sx)) + (my + jnp.log(sy))
        _bin(kl, lab0_ref[0], s0_ref, c0_ref)

    @pl.when(i >= 8)
    def _scale1():
        x = x1_ref[0]
        y = y1_ref[0]
        h = 1024

        def halves(v):
            return v[:, :h], v[:, h:]

        mxa, mxb = halves(jnp.max(x, axis=0, keepdims=True))
        mx = jnp.maximum(mxa, mxb)
        mxf = jnp.concatenate([mx, mx], axis=1)
        ex = jnp.exp(x - mxf)
        sxa, sxb = halves(jnp.sum(ex, axis=0, keepdims=True))
        sx = sxa + sxb
        mya, myb = halves(jnp.max(y, axis=0, keepdims=True))
        my = jnp.maximum(mya, myb)
        myf = jnp.concatenate([my, my], axis=1)
        ey = jnp.exp(y - myf)
        sya, syb = halves(jnp.sum(ey, axis=0, keepdims=True))
        sy = sya + syb
        ta, tb = halves(jnp.sum(ex * (x - y), axis=0, keepdims=True))
        t = (ta + tb) / sx
        kl = t - (mx + jnp.log(sx)) + (my + jnp.log(sy))
        _bin(kl, lab1_ref[0], s1_ref, c1_ref)

    @pl.when(i == pl.num_programs(0) - 1)
    def _combine():
        gate = gate_ref[:, :1]

        def term(s_ref, c_ref):
            s = jnp.sum(s_ref[...], axis=1, keepdims=True)
            c = jnp.sum(c_ref[...], axis=1, keepdims=True)
            klc = s / jnp.maximum(c, 1.0)
            return jnp.sum(gate * jnp.where(c > 0, klc, jnp.float32(0.0)))

        loss = term(s0_ref, c0_ref) + jnp.float32(2.0) * term(s1_ref, c1_ref)
        out_ref[...] = jnp.full((8, LANES), loss, jnp.float32)


def kernel(pseudo_labels, feat_old_0, feat_0, feat_old_1, feat_1, num_class, num_old_class):
    b = pseudo_labels.shape[0]

    lab0 = pseudo_labels[:, 0, ::8, ::8].reshape(2 * b, 1, 2048)
    lab1 = pseudo_labels[:, 0, ::16, ::16].reshape(b, 1, 1024)

    x0 = feat_0.reshape(b, 384, 4096)
    y0 = feat_old_0.reshape(b, 384, 4096)
    x1 = feat_1.reshape(b, 384, 2048)
    y1 = feat_old_1.reshape(b, 384, 2048)

    cls = jnp.arange(NCLS, dtype=jnp.float32)
    noc = jnp.asarray(num_old_class, jnp.float32)
    nc = jnp.asarray(num_class, jnp.float32)
    gate = jnp.where(
        cls == 0,
        noc / nc,
        jnp.where((cls <= noc) & (cls < 21), jnp.float32(1.0), jnp.float32(0.0)),
    )
    gate2d = jnp.broadcast_to(gate[:, None], (NCLS, LANES))

    def pin7(i):
        j = jnp.minimum(i, 7)
        return (j // 2, 0, j % 2)

    def adv1(i):
        return (jnp.maximum(i - 8, 0), 0, 0)

    grid = (12,)
    out = pl.pallas_call(
        _body,
        grid=grid,
        in_specs=[
            pl.BlockSpec((NCLS, LANES), lambda i: (0, 0)),
            pl.BlockSpec((1, 384, 2048), pin7),
            pl.BlockSpec((1, 384, 2048), pin7),
            pl.BlockSpec((1, 1, 2048), lambda i: (jnp.minimum(i, 7), 0, 0)),
            pl.BlockSpec((1, 384, 2048), adv1),
            pl.BlockSpec((1, 384, 2048), adv1),
            pl.BlockSpec((1, 1, 1024), adv1),
        ],
        out_specs=pl.BlockSpec((8, LANES), lambda i: (0, 0)),
        out_shape=jax.ShapeDtypeStruct((8, LANES), jnp.float32),
        scratch_shapes=[pltpu.VMEM((NCLS, LANES), jnp.float32)] * 4,
    )(gate2d, x0, y0, lab0, x1, y1, lab1)
    return out[0, 0]
